# baseline (speedup 1.0000x reference)
import jax
import jax.numpy as jnp
from jax.experimental import pallas as pl
from jax.experimental.pallas import tpu as pltpu

B = 16
D = 64
HW = 1024
N = 1024
BETA = 0.25
BPG = 16


def _vq_body(z_ref, w_ref, zq_ref, idx_ref, acc_ref, div_ref):
    g = pl.program_id(0)
    w = w_ref[...]
    wsq = jnp.sum(w * w, axis=1, keepdims=True)
    w2 = -2.0 * w
    vals = []
    dvals = []
    for i in range(BPG):
        zc = z_ref[i]
        dots2 = jax.lax.dot_general(
            w2, zc, (((1,), (0,)), ((), ())),
            preferred_element_type=jnp.float32)
        dist_t = wsq + dots2
        min_d = jnp.min(dist_t, axis=0, keepdims=True)
        iota_t = jax.lax.broadcasted_iota(jnp.int32, (N, HW), 0)
        keys = jax.lax.bitcast_convert_type(
            iota_t + jnp.int32(0x4B000000), jnp.float32)
        sentinel = jax.lax.bitcast_convert_type(
            jnp.int32(0x4B000000 + N), jnp.float32)
        keymin = jnp.min(jnp.where(dist_t == min_d, keys, sentinel),
                         axis=0, keepdims=True)
        idx = (jax.lax.bitcast_convert_type(keymin, jnp.int32)
               - jnp.int32(0x4B000000))[0]
        idx_ref[i, 0] = idx
        ohf = (keys == keymin).astype(jnp.float32)
        cnts = jax.lax.dot_general(
            ohf, jnp.ones((HW, 128), jnp.float32),
            (((1,), (0,)), ((), ())),
            preferred_element_type=jnp.float32)
        usedf = (cnts[:, 0:1] > 0.0).astype(jnp.float32)
        dvals.append(jnp.sum(usedf))
        zq = jax.lax.dot_general(
            w, ohf, (((0,), (0,)), ((), ())),
            preferred_element_type=jnp.float32)
        zq_ref[i] = zq
        vals.append(jnp.sum(min_d) + jnp.sum(zc * zc))
    val = sum(vals)
    dval = sum(dvals)

    @pl.when(g == 0)
    def _():
        acc_ref[0, 0] = val
        div_ref[0, 0] = dval

    @pl.when(g > 0)
    def _():
        acc_ref[0, 0] += val
        div_ref[0, 0] += dval


def kernel(z, weight):
    zr = z.reshape(B, D, HW)
    zq, idx, acc, div = pl.pallas_call(
        _vq_body,
        grid=(B // BPG,),
        in_specs=[
            pl.BlockSpec((BPG, D, HW), lambda b: (b, 0, 0)),
            pl.BlockSpec((N, D), lambda b: (0, 0)),
        ],
        out_specs=[
            pl.BlockSpec((BPG, D, HW), lambda b: (b, 0, 0)),
            pl.BlockSpec((BPG, 1, HW), lambda b: (b, 0, 0)),
            pl.BlockSpec(memory_space=pltpu.SMEM),
            pl.BlockSpec(memory_space=pltpu.SMEM),
        ],
        out_shape=[
            jax.ShapeDtypeStruct((B, D, HW), jnp.float32),
            jax.ShapeDtypeStruct((B, 1, HW), jnp.int32),
            jax.ShapeDtypeStruct((1, 1), jnp.float32),
            jax.ShapeDtypeStruct((1, 1), jnp.float32),
        ],
        compiler_params=pltpu.CompilerParams(
            dimension_semantics=("arbitrary",),
        ),
    )(zr, weight)
    z_q_out = zq.reshape(B, D, 32, 32)
    index = idx.reshape(B, 32, 32)
    loss = acc[0, 0] * ((1.0 + BETA) / (B * HW * D))
    diversity = div[0, 0] / (B * HW)
    return z_q_out, index, loss, diversity


# speedup vs baseline: 1.0234x; 1.0234x over previous
import jax
import jax.numpy as jnp
from jax.experimental import pallas as pl
from jax.experimental.pallas import tpu as pltpu

B = 16
D = 64
HW = 1024
N = 1024
BETA = 0.25
BPG = 8


def _vq_body(z_ref, w_ref, zq_ref, idx_ref, acc_ref, div_ref):
    g = pl.program_id(0)
    w = w_ref[...]
    wsq = jnp.sum(w * w, axis=1, keepdims=True)
    w2 = -2.0 * w
    iota_t = jax.lax.broadcasted_iota(jnp.int32, (N, HW), 0)
    keys = jax.lax.bitcast_convert_type(
        iota_t + jnp.int32(0x4B000000), jnp.float32)
    sentinel = jax.lax.bitcast_convert_type(
        jnp.int32(0x4B000000 + N), jnp.float32)
    vals = []
    dvals = []
    for i in range(BPG):
        zc = z_ref[i]
        dots2 = jax.lax.dot_general(
            w2, zc, (((1,), (0,)), ((), ())),
            preferred_element_type=jnp.float32)
        dist_t = wsq + dots2
        min_d = jnp.min(dist_t, axis=0, keepdims=True)
        keymin = jnp.min(jnp.where(dist_t == min_d, keys, sentinel),
                         axis=0, keepdims=True)
        idx = (jax.lax.bitcast_convert_type(keymin, jnp.int32)
               - jnp.int32(0x4B000000))[0]
        idx_ref[i, 0] = idx
        ohf = (keys == keymin).astype(jnp.float32)
        cnts = jax.lax.dot_general(
            ohf, jnp.ones((HW, 128), jnp.float32),
            (((1,), (0,)), ((), ())),
            preferred_element_type=jnp.float32)
        usedf = (cnts[:, 0:1] > 0.0).astype(jnp.float32)
        dvals.append(jnp.sum(usedf))
        zq = jax.lax.dot_general(
            w, ohf, (((0,), (0,)), ((), ())),
            preferred_element_type=jnp.float32)
        zq_ref[i] = zq
        vals.append(jnp.sum(min_d) + jnp.sum(zc * zc))
    val = sum(vals)
    dval = sum(dvals)

    @pl.when(g == 0)
    def _():
        acc_ref[0, 0] = val
        div_ref[0, 0] = dval

    @pl.when(g > 0)
    def _():
        acc_ref[0, 0] += val
        div_ref[0, 0] += dval


def kernel(z, weight):
    zr = z.reshape(B, D, HW)
    zq, idx, acc, div = pl.pallas_call(
        _vq_body,
        grid=(B // BPG,),
        in_specs=[
            pl.BlockSpec((BPG, D, HW), lambda b: (b, 0, 0)),
            pl.BlockSpec((N, D), lambda b: (0, 0)),
        ],
        out_specs=[
            pl.BlockSpec((BPG, D, HW), lambda b: (b, 0, 0)),
            pl.BlockSpec((BPG, 1, HW), lambda b: (b, 0, 0)),
            pl.BlockSpec(memory_space=pltpu.SMEM),
            pl.BlockSpec(memory_space=pltpu.SMEM),
        ],
        out_shape=[
            jax.ShapeDtypeStruct((B, D, HW), jnp.float32),
            jax.ShapeDtypeStruct((B, 1, HW), jnp.int32),
            jax.ShapeDtypeStruct((1, 1), jnp.float32),
            jax.ShapeDtypeStruct((1, 1), jnp.float32),
        ],
        compiler_params=pltpu.CompilerParams(
            dimension_semantics=("arbitrary",),
        ),
    )(zr, weight)
    z_q_out = zq.reshape(B, D, 32, 32)
    index = idx.reshape(B, 32, 32)
    loss = acc[0, 0] * ((1.0 + BETA) / (B * HW * D))
    diversity = div[0, 0] / (B * HW)
    return z_q_out, index, loss, diversity
